# baseline (speedup 1.0000x reference)
import functools

import jax
import jax.numpy as jnp
from jax import lax
from jax.experimental import pallas as pl
from jax.experimental.pallas import tpu as pltpu
from jax.experimental.pallas import tpu_sc as plsc

N = 10000
D = 128
E = 320000
NC = 2
NS = 16
CHUNK = 125
NCHUNK = 80
NSLAB = 10
SLAB = N // NSLAB
EPS = 1e-16

_MESH = plsc.VectorSubcoreMesh(core_axis_name="c", subcore_axis_name="s")


@functools.partial(
    pl.kernel,
    out_type=jax.ShapeDtypeStruct((NC, N), jnp.float32),
    mesh=_MESH,
    scratch_types=[
        pltpu.VMEM((NCHUNK, CHUNK), jnp.int32),
        pltpu.VMEM((128,), jnp.float32),
        pltpu.VMEM_SHARED((N,), jnp.float32),
    ],
)
def _deg_kernel(src_hbm, zeros1_hbm, deg_hbm, idx_v, ones_v, acc_sh):
    cid = lax.axis_index("c")
    sid = lax.axis_index("s")
    pltpu.sync_copy(src_hbm.at[cid, sid], idx_v)
    for k in range(8):
        ones_v[pl.ds(k * 16, 16)] = jnp.ones((16,), jnp.float32)

    @pl.when(sid == 0)
    def _():
        pltpu.sync_copy(zeros1_hbm, acc_sh)

    plsc.subcore_barrier()

    def body(j, carry):
        pltpu.sync_copy(ones_v.at[pl.ds(0, CHUNK)],
                        acc_sh.at[idx_v.at[j]], add=True)
        return carry

    lax.fori_loop(0, NCHUNK, body, 0)
    plsc.subcore_barrier()

    @pl.when(sid == 0)
    def _():
        pltpu.sync_copy(acc_sh, deg_hbm.at[cid])


@functools.partial(
    pl.kernel,
    out_type=jax.ShapeDtypeStruct((NC, NSLAB, SLAB, D), jnp.float32),
    mesh=_MESH,
    scratch_types=[
        pltpu.VMEM((NCHUNK // 2, CHUNK), jnp.int32),
        pltpu.VMEM((NCHUNK // 2, CHUNK), jnp.int32),
        pltpu.VMEM((CHUNK, D), jnp.float32),
        pltpu.VMEM((CHUNK, D), jnp.float32),
        pltpu.VMEM_SHARED((N, D), jnp.float32),
        pltpu.SemaphoreType.DMA,
        pltpu.SemaphoreType.DMA,
        pltpu.SemaphoreType.DMA,
        pltpu.SemaphoreType.DMA,
    ],
)
def _scatter_pass(x_hbm, src_hbm, dst_hbm, zeros2_hbm, out_hbm,
                  sidx_v, didx_v, buf0, buf1, acc_sh, g0, g1, s0, s1):
    cid = lax.axis_index("c")
    sid = lax.axis_index("s")
    NH = NCHUNK // 2

    @pl.when(sid < NSLAB)
    def _():
        row0 = sid * SLAB
        pltpu.sync_copy(zeros2_hbm.at[pl.ds(row0, SLAB)],
                        acc_sh.at[pl.ds(row0, SLAB)])

    plsc.subcore_barrier()

    for h in range(2):
        pltpu.sync_copy(src_hbm.at[cid, sid, pl.ds(h * NH, NH)], sidx_v)
        pltpu.sync_copy(dst_hbm.at[cid, sid, pl.ds(h * NH, NH)], didx_v)
        pltpu.async_copy(x_hbm.at[sidx_v.at[0]], buf0, g0)

        def body(j, carry):
            @pl.when(j % 2 == 0)
            def _():
                pltpu.make_async_copy(x_hbm.at[sidx_v.at[j]], buf0,
                                      g0).wait()

                @pl.when(j >= 1)
                def _():
                    pltpu.make_async_copy(buf1, acc_sh.at[didx_v.at[j - 1]],
                                          s1).wait()

                @pl.when(j + 1 < NH)
                def _():
                    pltpu.async_copy(x_hbm.at[sidx_v.at[j + 1]], buf1, g1)

                pltpu.async_copy(buf0, acc_sh.at[didx_v.at[j]], s0, add=True)

            @pl.when(j % 2 == 1)
            def _():
                pltpu.make_async_copy(x_hbm.at[sidx_v.at[j]], buf1,
                                      g1).wait()
                pltpu.make_async_copy(buf0, acc_sh.at[didx_v.at[j - 1]],
                                      s0).wait()

                @pl.when(j + 1 < NH)
                def _():
                    pltpu.async_copy(x_hbm.at[sidx_v.at[j + 1]], buf0, g0)

                pltpu.async_copy(buf1, acc_sh.at[didx_v.at[j]], s1, add=True)

            return carry

        lax.fori_loop(0, NH, body, 0)
        pltpu.make_async_copy(buf1, acc_sh.at[didx_v.at[NH - 1]], s1).wait()

    plsc.subcore_barrier()

    @pl.when(sid < NSLAB)
    def _():
        row0 = sid * SLAB
        pltpu.sync_copy(acc_sh.at[pl.ds(row0, SLAB)], out_hbm.at[cid, sid])


BN = 2000


def _scale1_body(degp_ref, x_ref, x1_ref):
    deg = degp_ref[:, 0] + degp_ref[:, 1]
    r = lax.rsqrt(deg + EPS)
    x1_ref[...] = x_ref[...] * r[:, None]


def _scale1(degp, x):
    return pl.pallas_call(
        _scale1_body,
        grid=(N // BN,),
        in_specs=[
            pl.BlockSpec((BN, NC), lambda i: (i, 0)),
            pl.BlockSpec((BN, D), lambda i: (i, 0)),
        ],
        out_specs=pl.BlockSpec((BN, D), lambda i: (i, 0)),
        out_shape=jax.ShapeDtypeStruct((N, D), jnp.float32),
    )(degp, x)


def _scale2_body(degp_ref, s1_ref, x3_ref):
    deg = degp_ref[:, 0] + degp_ref[:, 1]
    r = lax.rsqrt(deg + EPS)
    x3_ref[...] = (s1_ref[0] + s1_ref[1]) * (r * r)[:, None]


def _scale2(degp, s1):
    return pl.pallas_call(
        _scale2_body,
        grid=(N // BN,),
        in_specs=[
            pl.BlockSpec((BN, NC), lambda i: (i, 0)),
            pl.BlockSpec((NC, BN, D), lambda i: (0, i, 0)),
        ],
        out_specs=pl.BlockSpec((BN, D), lambda i: (i, 0)),
        out_shape=jax.ShapeDtypeStruct((N, D), jnp.float32),
    )(degp, s1)


def _final_body(degp_ref, s2_ref, w0_ref, wm_ref, ws_ref, wp_ref, wa_ref,
                dm_ref, ds_ref, dp_ref, da_ref,
                bm_ref, bs_ref, bp_ref, ba_ref,
                zm_ref, zs_ref, pi_ref, al_ref):
    deg = degp_ref[:, 0] + degp_ref[:, 1]
    r = lax.rsqrt(deg + EPS)
    q = (s2_ref[0] + s2_ref[1]) * r[:, None]
    p = jnp.dot(q, w0_ref[...], preferred_element_type=jnp.float32)

    def head(w_ref, d_ref, b_ref):
        h = jax.nn.sigmoid(
            jnp.dot(p, w_ref[...], preferred_element_type=jnp.float32))
        return jnp.dot(h, d_ref[...],
                       preferred_element_type=jnp.float32) + b_ref[...]

    zm_ref[...] = head(wm_ref, dm_ref, bm_ref)
    zs_ref[...] = head(ws_ref, ds_ref, bs_ref)
    pi_ref[...] = head(wp_ref, dp_ref, bp_ref)
    t = head(wa_ref, da_ref, ba_ref)
    al_ref[...] = jnp.maximum(t, 0.0) + jnp.log1p(jnp.exp(-jnp.abs(t)))


def _final(degp, s2, W0, Wm, Ws, Wp, Wa, Dm, Ds, Dp, Da, bm, bs, bp, ba):
    h2, do = Wm.shape[1], Dm.shape[1]
    wspec = pl.BlockSpec((D, D), lambda i: (0, 0))
    bspec = pl.BlockSpec((D, h2), lambda i: (0, 0))
    dspec = pl.BlockSpec((h2, do), lambda i: (0, 0))
    vspec = pl.BlockSpec((1, do), lambda i: (0, 0))
    ospec = pl.BlockSpec((BN, do), lambda i: (i, 0))
    oshape = jax.ShapeDtypeStruct((N, do), jnp.float32)
    return pl.pallas_call(
        _final_body,
        grid=(N // BN,),
        in_specs=[
            pl.BlockSpec((BN, NC), lambda i: (i, 0)),
            pl.BlockSpec((NC, BN, D), lambda i: (0, i, 0)),
            wspec, bspec, bspec, bspec, bspec,
            dspec, dspec, dspec, dspec,
            vspec, vspec, vspec, vspec,
        ],
        out_specs=[ospec, ospec, ospec, ospec],
        out_shape=[oshape, oshape, oshape, oshape],
    )(degp, s2, W0, Wm, Ws, Wp, Wa, Dm, Ds, Dp, Da, bm, bs, bp, ba)


def kernel(x, edge_index, W0, Wm, Ws, Wp, Wa, Dm, bm, Ds, bs, Dp, bp, Da, ba):
    src = edge_index[0].astype(jnp.int32).reshape(NC, NS, NCHUNK, CHUNK)
    dst = edge_index[1].astype(jnp.int32).reshape(NC, NS, NCHUNK, CHUNK)
    zeros1 = jnp.zeros((N,), jnp.float32)
    zeros2 = jnp.zeros((N, D), jnp.float32)

    degp = _deg_kernel(src, zeros1).T
    x1 = _scale1(degp, x)
    s1 = _scatter_pass(x1, src, dst, zeros2).reshape(NC, N, D)
    x3 = _scale2(degp, s1)
    s2 = _scatter_pass(x3, src, dst, zeros2).reshape(NC, N, D)
    return _final(degp, s2, W0, Wm, Ws, Wp, Wa, Dm, Ds, Dp, Da,
                  bm.reshape(1, -1), bs.reshape(1, -1),
                  bp.reshape(1, -1), ba.reshape(1, -1))


# speedup vs baseline: 1.1842x; 1.1842x over previous
import functools

import jax
import jax.numpy as jnp
from jax import lax
from jax.experimental import pallas as pl
from jax.experimental.pallas import tpu as pltpu
from jax.experimental.pallas import tpu_sc as plsc

N = 10000
D = 128
E = 320000
NC = 2
NS = 16
CHUNK = 125
NCHUNK = 80
NSLAB = 10
SLAB = N // NSLAB
EPS = 1e-16

_MESH = plsc.VectorSubcoreMesh(core_axis_name="c", subcore_axis_name="s")


@functools.partial(
    pl.kernel,
    out_type=jax.ShapeDtypeStruct((NC, N), jnp.float32),
    mesh=_MESH,
    scratch_types=[
        pltpu.VMEM((NCHUNK, CHUNK), jnp.int32),
        pltpu.VMEM((128,), jnp.float32),
        pltpu.VMEM_SHARED((N,), jnp.float32),
    ],
)
def _deg_kernel(src_hbm, zeros1_hbm, deg_hbm, idx_v, ones_v, acc_sh):
    cid = lax.axis_index("c")
    sid = lax.axis_index("s")
    pltpu.sync_copy(src_hbm.at[cid, sid], idx_v)
    for k in range(8):
        ones_v[pl.ds(k * 16, 16)] = jnp.ones((16,), jnp.float32)

    @pl.when(sid == 0)
    def _():
        pltpu.sync_copy(zeros1_hbm, acc_sh)

    plsc.subcore_barrier()

    def body(j, carry):
        pltpu.sync_copy(ones_v.at[pl.ds(0, CHUNK)],
                        acc_sh.at[idx_v.at[j]], add=True)
        return carry

    lax.fori_loop(0, NCHUNK, body, 0)
    plsc.subcore_barrier()

    @pl.when(sid == 0)
    def _():
        pltpu.sync_copy(acc_sh, deg_hbm.at[cid])


@functools.partial(
    pl.kernel,
    out_type=jax.ShapeDtypeStruct((NC, NSLAB, SLAB, D), jnp.float32),
    mesh=_MESH,
    scratch_types=[
        pltpu.VMEM((NCHUNK // 2, CHUNK), jnp.int32),
        pltpu.VMEM((NCHUNK // 2, CHUNK), jnp.int32),
        pltpu.VMEM((CHUNK, D), jnp.float32),
        pltpu.VMEM((CHUNK, D), jnp.float32),
        pltpu.VMEM_SHARED((N, D), jnp.float32),
        pltpu.SemaphoreType.DMA,
        pltpu.SemaphoreType.DMA,
        pltpu.SemaphoreType.DMA,
        pltpu.SemaphoreType.DMA,
    ],
)
def _scatter_pass(x_hbm, src_hbm, dst_hbm, zeros2_hbm, out_hbm,
                  sidx_v, didx_v, buf0, buf1, acc_sh, g0, g1, s0, s1):
    cid = lax.axis_index("c")
    sid = lax.axis_index("s")
    NH = NCHUNK // 2

    @pl.when(sid < NSLAB)
    def _():
        row0 = sid * SLAB
        pltpu.sync_copy(zeros2_hbm.at[pl.ds(row0, SLAB)],
                        acc_sh.at[pl.ds(row0, SLAB)])

    plsc.subcore_barrier()

    for h in range(2):
        pltpu.sync_copy(src_hbm.at[cid, sid, pl.ds(h * NH, NH)], sidx_v)
        pltpu.sync_copy(dst_hbm.at[cid, sid, pl.ds(h * NH, NH)], didx_v)
        pltpu.async_copy(x_hbm.at[sidx_v.at[0]], buf0, g0)

        def body(j, carry):
            @pl.when(j % 2 == 0)
            def _():
                @pl.when(j + 1 < NH)
                def _():
                    pltpu.async_copy(x_hbm.at[sidx_v.at[j + 1]], buf1, g1)

                pltpu.make_async_copy(x_hbm.at[sidx_v.at[j]], buf0,
                                      g0).wait()
                pltpu.sync_copy(buf0, acc_sh.at[didx_v.at[j]], add=True)

            @pl.when(j % 2 == 1)
            def _():
                @pl.when(j + 1 < NH)
                def _():
                    pltpu.async_copy(x_hbm.at[sidx_v.at[j + 1]], buf0, g0)

                pltpu.make_async_copy(x_hbm.at[sidx_v.at[j]], buf1,
                                      g1).wait()
                pltpu.sync_copy(buf1, acc_sh.at[didx_v.at[j]], add=True)

            return carry

        lax.fori_loop(0, NH, body, 0)

    plsc.subcore_barrier()

    @pl.when(sid < NSLAB)
    def _():
        row0 = sid * SLAB
        pltpu.sync_copy(acc_sh.at[pl.ds(row0, SLAB)], out_hbm.at[cid, sid])


BN = 2000


def _scale1_body(degp_ref, x_ref, x1_ref):
    deg = degp_ref[0, :] + degp_ref[1, :]
    r = lax.rsqrt(deg + EPS)
    x1_ref[...] = x_ref[...] * r[:, None]


def _scale1(degp, x):
    return pl.pallas_call(
        _scale1_body,
        out_shape=jax.ShapeDtypeStruct((N, D), jnp.float32),
    )(degp, x)


def _scale2_body(degp_ref, s1_ref, x3_ref):
    deg = degp_ref[0, :] + degp_ref[1, :]
    r = lax.rsqrt(deg + EPS)
    x3_ref[...] = (s1_ref[0] + s1_ref[1]) * (r * r)[:, None]


def _scale2(degp, s1):
    return pl.pallas_call(
        _scale2_body,
        out_shape=jax.ShapeDtypeStruct((N, D), jnp.float32),
    )(degp, s1)


def _final_body(degp_ref, s2_ref, w0_ref, wm_ref, ws_ref, wp_ref, wa_ref,
                dm_ref, ds_ref, dp_ref, da_ref,
                bm_ref, bs_ref, bp_ref, ba_ref,
                zm_ref, zs_ref, pi_ref, al_ref):
    i = pl.program_id(0)
    degb = degp_ref[:, pl.ds(pl.multiple_of(i * BN, 128), BN)]
    deg = degb[0, :] + degb[1, :]
    r = lax.rsqrt(deg + EPS)
    q = (s2_ref[0] + s2_ref[1]) * r[:, None]
    p = jnp.dot(q, w0_ref[...], preferred_element_type=jnp.float32)

    def head(w_ref, d_ref, b_ref):
        h = jax.nn.sigmoid(
            jnp.dot(p, w_ref[...], preferred_element_type=jnp.float32))
        return jnp.dot(h, d_ref[...],
                       preferred_element_type=jnp.float32) + b_ref[...]

    zm_ref[...] = head(wm_ref, dm_ref, bm_ref)
    zs_ref[...] = head(ws_ref, ds_ref, bs_ref)
    pi_ref[...] = head(wp_ref, dp_ref, bp_ref)
    t = head(wa_ref, da_ref, ba_ref)
    al_ref[...] = jnp.maximum(t, 0.0) + jnp.log1p(jnp.exp(-jnp.abs(t)))


def _final(degp, s2, W0, Wm, Ws, Wp, Wa, Dm, Ds, Dp, Da, bm, bs, bp, ba):
    h2, do = Wm.shape[1], Dm.shape[1]
    wspec = pl.BlockSpec((D, D), lambda i: (0, 0))
    bspec = pl.BlockSpec((D, h2), lambda i: (0, 0))
    dspec = pl.BlockSpec((h2, do), lambda i: (0, 0))
    vspec = pl.BlockSpec((1, do), lambda i: (0, 0))
    ospec = pl.BlockSpec((BN, do), lambda i: (i, 0))
    oshape = jax.ShapeDtypeStruct((N, do), jnp.float32)
    return pl.pallas_call(
        _final_body,
        grid=(N // BN,),
        in_specs=[
            pl.BlockSpec((NC, N), lambda i: (0, 0)),
            pl.BlockSpec((NC, BN, D), lambda i: (0, i, 0)),
            wspec, bspec, bspec, bspec, bspec,
            dspec, dspec, dspec, dspec,
            vspec, vspec, vspec, vspec,
        ],
        out_specs=[ospec, ospec, ospec, ospec],
        out_shape=[oshape, oshape, oshape, oshape],
    )(degp, s2, W0, Wm, Ws, Wp, Wa, Dm, Ds, Dp, Da, bm, bs, bp, ba)


def kernel(x, edge_index, W0, Wm, Ws, Wp, Wa, Dm, bm, Ds, bs, Dp, bp, Da, ba):
    src = edge_index[0].astype(jnp.int32).reshape(NC, NS, NCHUNK, CHUNK)
    dst = edge_index[1].astype(jnp.int32).reshape(NC, NS, NCHUNK, CHUNK)
    zeros1 = jnp.zeros((N,), jnp.float32)
    zeros2 = jnp.zeros((N, D), jnp.float32)

    degp = _deg_kernel(src, zeros1)
    x1 = _scale1(degp, x)
    s1 = _scatter_pass(x1, src, dst, zeros2).reshape(NC, N, D)
    x3 = _scale2(degp, s1)
    s2 = _scatter_pass(x3, src, dst, zeros2).reshape(NC, N, D)
    return _final(degp, s2, W0, Wm, Ws, Wp, Wa, Dm, Ds, Dp, Da,
                  bm.reshape(1, -1), bs.reshape(1, -1),
                  bp.reshape(1, -1), ba.reshape(1, -1))
